# baseline (speedup 1.0000x reference)
import jax
import jax.numpy as jnp
from jax.experimental import pallas as pl
from jax.experimental.pallas import tpu as pltpu

N = 32768
D = 256
D1 = 128
K1 = 64
KPC = 8
K2 = K1 * KPC
BN = 2048
BIG = 2.0 ** 20


def _outer_kernel(x_hbm, c1_ref, n1t_ref, c2m_ref, n2t_ref, out_hbm):
    def body(x_ref, o_ref):
        x = x_ref[...]
        x1 = x[:, :D1]
        x2 = x[:, D1:]
        sq = x1 * x1
        xn1t = jax.lax.dot_general(jnp.ones((1, D1), jnp.float32), sq,
                                   (((1,), (1,)), ((), ())))
        mm1 = jax.lax.dot_general(c1_ref[...], x1,
                                  (((1,), (1,)), ((), ())))
        d1t = xn1t - 2.0 * mm1 + n1t_ref[...]
        outer = jnp.argmin(d1t, axis=0).astype(jnp.int32)
        mm2 = jax.lax.dot_general(c2m_ref[...], x2.astype(jnp.bfloat16),
                                  (((1,), (1,)), ((), ())),
                                  preferred_element_type=jnp.float32)
        d2b = mm2.astype(jnp.bfloat16) + n2t_ref[...]
        kcol = jax.lax.broadcasted_iota(jnp.int32, (K2, 1), 0)
        grp_col = (kcol >> 3).astype(jnp.bfloat16)
        loc_col = (kcol & 7).astype(jnp.bfloat16)
        outer_b = outer.astype(jnp.bfloat16)
        d2m = jnp.where(grp_col == outer_b[None, :], d2b, jnp.bfloat16(BIG))
        mval = jnp.min(d2m, axis=0)
        ikey = jnp.where(d2m == mval[None, :], loc_col, jnp.bfloat16(15.0))
        inner = jnp.min(ikey, axis=0).astype(jnp.int32)
        o_ref[0, 0, :] = outer * KPC + inner

    pltpu.emit_pipeline(
        body,
        grid=(N // BN,),
        in_specs=[pl.BlockSpec((BN, D), lambda i: (i, 0))],
        out_specs=[pl.BlockSpec((1, 1, BN), lambda i: (i, 0, 0))],
    )(x_hbm, out_hbm)


def kernel(x, centers1, centers2):
    c2f = centers2.reshape(K2, D1)
    n1t = jnp.sum(centers1 * centers1, axis=1)[:, None]
    n2 = jnp.sum(c2f * c2f, axis=1)
    c2m = (-2.0 * c2f).astype(jnp.bfloat16)
    n2t = n2[:, None].astype(jnp.bfloat16)
    grid = N // BN
    out = pl.pallas_call(
        _outer_kernel,
        in_specs=[
            pl.BlockSpec(memory_space=pltpu.MemorySpace.HBM),
            pl.BlockSpec((K1, D1), lambda: (0, 0)),
            pl.BlockSpec((K1, 1), lambda: (0, 0)),
            pl.BlockSpec((K2, D1), lambda: (0, 0)),
            pl.BlockSpec((K2, 1), lambda: (0, 0)),
        ],
        out_specs=pl.BlockSpec(memory_space=pltpu.MemorySpace.HBM),
        out_shape=jax.ShapeDtypeStruct((grid, 1, BN), jnp.int32),
    )(x, centers1, n1t, c2m, n2t)
    return out.reshape(N)


# speedup vs baseline: 1.0907x; 1.0907x over previous
import jax
import jax.numpy as jnp
from jax.experimental import pallas as pl
from jax.experimental.pallas import tpu as pltpu

N = 32768
D = 256
D1 = 128
K1 = 64
KPC = 8
K2 = K1 * KPC
BN = 4096
BIG = 2.0 ** 20


def _outer_kernel(x_hbm, c1_ref, n1t_ref, c2m_ref, n2t_ref, out_hbm):
    def body(x_ref, o_ref):
        x = x_ref[...]
        x1 = x[:, :D1]
        x2 = x[:, D1:]
        sq = x1 * x1
        xn1t = jax.lax.dot_general(jnp.ones((1, D1), jnp.float32), sq,
                                   (((1,), (1,)), ((), ())))
        mm1 = jax.lax.dot_general(c1_ref[...], x1,
                                  (((1,), (1,)), ((), ())))
        d1t = xn1t - 2.0 * mm1 + n1t_ref[...]
        outer = jnp.argmin(d1t, axis=0).astype(jnp.int32)
        mm2 = jax.lax.dot_general(c2m_ref[...], x2.astype(jnp.bfloat16),
                                  (((1,), (1,)), ((), ())),
                                  preferred_element_type=jnp.float32)
        d2b = mm2.astype(jnp.bfloat16) + n2t_ref[...]
        kcol = jax.lax.broadcasted_iota(jnp.int32, (K2, 1), 0)
        grp_col = (kcol >> 3).astype(jnp.bfloat16)
        loc_col = (kcol & 7).astype(jnp.bfloat16)
        outer_b = outer.astype(jnp.bfloat16)
        d2m = jnp.where(grp_col == outer_b[None, :], d2b, jnp.bfloat16(BIG))
        mval = jnp.min(d2m, axis=0)
        ikey = jnp.where(d2m == mval[None, :], loc_col, jnp.bfloat16(15.0))
        inner = jnp.min(ikey, axis=0).astype(jnp.int32)
        o_ref[0, 0, :] = outer * KPC + inner

    pltpu.emit_pipeline(
        body,
        grid=(N // BN,),
        in_specs=[pl.BlockSpec((BN, D), lambda i: (i, 0))],
        out_specs=[pl.BlockSpec((1, 1, BN), lambda i: (i, 0, 0))],
    )(x_hbm, out_hbm)


def kernel(x, centers1, centers2):
    c2f = centers2.reshape(K2, D1)
    n1t = jnp.sum(centers1 * centers1, axis=1)[:, None]
    n2 = jnp.sum(c2f * c2f, axis=1)
    c2m = (-2.0 * c2f).astype(jnp.bfloat16)
    n2t = n2[:, None].astype(jnp.bfloat16)
    grid = N // BN
    out = pl.pallas_call(
        _outer_kernel,
        in_specs=[
            pl.BlockSpec(memory_space=pltpu.MemorySpace.HBM),
            pl.BlockSpec((K1, D1), lambda: (0, 0)),
            pl.BlockSpec((K1, 1), lambda: (0, 0)),
            pl.BlockSpec((K2, D1), lambda: (0, 0)),
            pl.BlockSpec((K2, 1), lambda: (0, 0)),
        ],
        out_specs=pl.BlockSpec(memory_space=pltpu.MemorySpace.HBM),
        out_shape=jax.ShapeDtypeStruct((grid, 1, BN), jnp.int32),
    )(x, centers1, n1t, c2m, n2t)
    return out.reshape(N)


# speedup vs baseline: 1.0992x; 1.0078x over previous
import jax
import jax.numpy as jnp
from jax.experimental import pallas as pl

N = 32768
D = 256
D1 = 128
K1 = 64
KPC = 8
K2 = K1 * KPC
BN = 8192
GA = 72
BIG = 2.0 ** 20


def _cluster_kernel(x_ref, c1_ref, n1t_ref, c2m_ref, n2t_ref, out_ref):
    x = x_ref[...]
    x1 = x[:, :D1]
    x2 = x[:, D1:]
    sq = x1 * x1
    xn1t = jax.lax.dot_general(jnp.ones((1, D1), jnp.float32), sq,
                               (((1,), (1,)), ((), ())))
    mm1 = jax.lax.dot_general(c1_ref[...], x1, (((1,), (1,)), ((), ())))
    d1t = xn1t - 2.0 * mm1 + n1t_ref[...]
    outer = jnp.argmin(d1t, axis=0).astype(jnp.int32)
    mm2 = jax.lax.dot_general(c2m_ref[...], x2.astype(jnp.bfloat16),
                              (((1,), (1,)), ((), ())),
                              preferred_element_type=jnp.float32)
    d2b = mm2.astype(jnp.bfloat16) + n2t_ref[...]
    kcol = jax.lax.broadcasted_iota(jnp.int32, (K2, 1), 0)
    grp_col = (kcol >> 3).astype(jnp.bfloat16)
    loc_col = (kcol & 7).astype(jnp.bfloat16)
    outer_b = outer.astype(jnp.bfloat16)
    d2m = jnp.where(grp_col == outer_b[None, :], d2b, jnp.bfloat16(BIG))
    mval = jnp.min(d2m, axis=0)
    ikey = jnp.where(d2m == mval[None, :], loc_col, jnp.bfloat16(15.0))
    inner = jnp.min(ikey, axis=0).astype(jnp.int32)
    out_ref[0, 0, :] = outer * KPC + inner


def kernel(x, centers1, centers2):
    c2f = centers2.reshape(K2, D1)
    n1t = jnp.sum(centers1 * centers1, axis=1)[:, None]
    n2 = jnp.sum(c2f * c2f, axis=1)
    c2m = (-2.0 * c2f).astype(jnp.bfloat16)
    n2t = n2[:, None].astype(jnp.bfloat16)
    grid = N // BN
    out = pl.pallas_call(
        _cluster_kernel,
        grid=(grid,),
        in_specs=[
            pl.BlockSpec((BN, D), lambda i: (i, 0)),
            pl.BlockSpec((K1, D1), lambda i: (0, 0)),
            pl.BlockSpec((K1, 1), lambda i: (0, 0)),
            pl.BlockSpec((K2, D1), lambda i: (0, 0)),
            pl.BlockSpec((K2, 1), lambda i: (0, 0)),
        ],
        out_specs=pl.BlockSpec((1, 1, BN), lambda i: (i, 0, 0)),
        out_shape=jax.ShapeDtypeStruct((grid, 1, BN), jnp.int32),
    )(x, centers1, n1t, c2m, n2t)
    return out.reshape(N)


# speedup vs baseline: 1.1476x; 1.0440x over previous
import jax
import jax.numpy as jnp
from jax.experimental import pallas as pl

N = 32768
D = 256
D1 = 128
K1 = 64
KPC = 8
K2 = K1 * KPC
BN = 4096
GA = 72
BIG = 2.0 ** 20


def _cluster_kernel(x_ref, c1_ref, n1t_ref, c2m_ref, n2t_ref, out_ref):
    x = x_ref[...]
    x1 = x[:, :D1]
    x2 = x[:, D1:]
    sq = x1 * x1
    xn1t = jax.lax.dot_general(jnp.ones((1, D1), jnp.float32), sq,
                               (((1,), (1,)), ((), ())))
    mm1 = jax.lax.dot_general(c1_ref[...], x1, (((1,), (1,)), ((), ())))
    d1t = xn1t - 2.0 * mm1 + n1t_ref[...]
    outer = jnp.argmin(d1t, axis=0).astype(jnp.int32)
    mm2 = jax.lax.dot_general(c2m_ref[...], x2.astype(jnp.bfloat16),
                              (((1,), (1,)), ((), ())),
                              preferred_element_type=jnp.float32)
    d2b = mm2.astype(jnp.bfloat16) + n2t_ref[...]
    kcol = jax.lax.broadcasted_iota(jnp.int32, (K2, 1), 0)
    grp_col = (kcol >> 3).astype(jnp.bfloat16)
    loc_col = (kcol & 7).astype(jnp.bfloat16)
    outer_b = outer.astype(jnp.bfloat16)
    d2m = jnp.where(grp_col == outer_b[None, :], d2b, jnp.bfloat16(BIG))
    mval = jnp.min(d2m, axis=0)
    ikey = jnp.where(d2m == mval[None, :], loc_col, jnp.bfloat16(15.0))
    inner = jnp.min(ikey, axis=0).astype(jnp.int32)
    out_ref[0, 0, :] = outer * KPC + inner


def kernel(x, centers1, centers2):
    c2f = centers2.reshape(K2, D1)
    n1t = jnp.sum(centers1 * centers1, axis=1)[:, None]
    n2 = jnp.sum(c2f * c2f, axis=1)
    c2m = (-2.0 * c2f).astype(jnp.bfloat16)
    n2t = n2[:, None].astype(jnp.bfloat16)
    grid = N // BN
    out = pl.pallas_call(
        _cluster_kernel,
        grid=(grid,),
        in_specs=[
            pl.BlockSpec((BN, D), lambda i: (i, 0)),
            pl.BlockSpec((K1, D1), lambda i: (0, 0)),
            pl.BlockSpec((K1, 1), lambda i: (0, 0)),
            pl.BlockSpec((K2, D1), lambda i: (0, 0)),
            pl.BlockSpec((K2, 1), lambda i: (0, 0)),
        ],
        out_specs=pl.BlockSpec((1, 1, BN), lambda i: (i, 0, 0)),
        out_shape=jax.ShapeDtypeStruct((grid, 1, BN), jnp.int32),
    )(x, centers1, n1t, c2m, n2t)
    return out.reshape(N)
